# baseline (speedup 1.0000x reference)
import functools

import jax
import jax.numpy as jnp
from jax import lax
from jax.experimental import pallas as pl
from jax.experimental.pallas import tpu as pltpu
from jax.experimental.pallas import tpu_sc as plsc

_BATCH = 16384
_FIELDS = 26
_DIM = 64
_TOTAL = _BATCH * _FIELDS
_NVOCAB = 1000000


def _detile_table(tab_t):
    cb = 16384
    grid = (_NVOCAB + cb - 1) // cb

    def body(in_ref, out_ref):
        out_ref[...] = in_ref[...].T

    return pl.pallas_call(
        body,
        grid=(grid,),
        in_specs=[pl.BlockSpec((_DIM, cb), lambda i: (0, i))],
        out_specs=pl.BlockSpec((cb, _DIM), lambda i: (i, 0)),
        out_shape=jax.ShapeDtypeStruct((_NVOCAB, _DIM), jnp.float32),
    )(tab_t)


def _build_gather():
    info = plsc.get_sparse_core_info()
    nc, ns = info.num_cores, info.num_subcores
    nw = nc * ns
    rows_per_w = _TOTAL // nw
    assert rows_per_w * nw == _TOTAL

    chunk = 256
    n_chunks = rows_per_w // chunk
    assert n_chunks * chunk == rows_per_w
    nbuf = 2
    n_groups = n_chunks // nbuf
    assert n_groups * nbuf == n_chunks
    g = 128
    n_g = chunk // g
    n_bg = chunk // 16

    mesh = plsc.VectorSubcoreMesh(core_axis_name="c", subcore_axis_name="s")

    @functools.partial(
        pl.kernel,
        mesh=mesh,
        out_type=jax.ShapeDtypeStruct((_FIELDS, _DIM, _BATCH), jnp.float32),
        scratch_types=[
            pltpu.VMEM((rows_per_w,), jnp.int32),
        ] + [pltpu.VMEM((chunk, _DIM), jnp.float32) for _ in range(nbuf)]
          + [pltpu.VMEM((_DIM, chunk), jnp.float32) for _ in range(nbuf)]
          + [pltpu.SemaphoreType.DMA for _ in range(2 * nbuf)],
        compiler_params=pltpu.CompilerParams(
            use_tc_tiling_on_sc=False, needs_layout_passes=False),
    )
    def emb(idx_hbm, table_hbm, out_hbm, idx_all, *bufs_and_sems):
        rows = bufs_and_sems[:nbuf]
        tbuf = bufs_and_sems[nbuf:2 * nbuf]
        sem_g = bufs_and_sems[2 * nbuf:3 * nbuf]
        sem_s = bufs_and_sems[3 * nbuf:]
        wid = lax.axis_index("s") * nc + lax.axis_index("c")
        base = wid * rows_per_w
        pltpu.sync_copy(idx_hbm.at[pl.ds(base, rows_per_w)], idx_all)
        lane = lax.broadcasted_iota(jnp.int32, (16,), 0)

        def store_wait(b):
            pltpu.make_async_copy(
                tbuf[b], out_hbm.at[0, :, pl.ds(0, chunk)], sem_s[b]
            ).wait()

        def group(i, carry):
            s = i * nbuf
            handles = []
            for b in range(nbuf):
                cstart = (s + b) * chunk
                hb = []
                for j in range(n_g):
                    hb.append(pltpu.async_copy(
                        table_hbm.at[idx_all.at[pl.ds(cstart + j * g, g)]],
                        rows[b].at[pl.ds(j * g, g)],
                        sem_g[b],
                    ))
                handles.append(hb)
            for b in range(nbuf):
                for c in handles[b]:
                    c.wait()

                @pl.when(i >= 1)
                def _():
                    store_wait(b)

                def tpose(bg, carry2):
                    row0 = bg * 16
                    for d in range(_DIM):
                        v = plsc.load_gather(
                            rows[b],
                            [row0 + lane, jnp.full((16,), d, jnp.int32)],
                        )
                        tbuf[b][d, pl.ds(row0, 16)] = v
                    return carry2

                lax.fori_loop(0, n_bg, tpose, 0)
                q0 = base + (s + b) * chunk
                f = q0 // _BATCH
                b0 = q0 - f * _BATCH
                pltpu.async_copy(
                    tbuf[b],
                    out_hbm.at[f, :, pl.ds(b0, chunk)],
                    sem_s[b],
                )
            return carry

        lax.fori_loop(0, n_groups, group, 0)
        for b in range(nbuf):
            store_wait(b)

    return emb


_emb = _build_gather()


def kernel(input, table):
    idx_q = input.T.reshape(_TOTAL)
    tab_t = table.T
    table_rm = _detile_table(tab_t)
    out_t = _emb(idx_q, table_rm)
    return out_t.transpose(2, 0, 1)


# speedup vs baseline: 1.4968x; 1.4968x over previous
import functools

import jax
import jax.numpy as jnp
from jax import lax
from jax.experimental import pallas as pl
from jax.experimental.pallas import tpu as pltpu
from jax.experimental.pallas import tpu_sc as plsc

_BATCH = 16384
_FIELDS = 26
_DIM = 64
_TOTAL = _BATCH * _FIELDS


def _build():
    info = plsc.get_sparse_core_info()
    nc, ns = info.num_cores, info.num_subcores
    nw = nc * ns
    rows_per_w = _TOTAL // nw
    assert rows_per_w * nw == _TOTAL

    nbuf = 4
    chunk = 256
    n_chunks = rows_per_w // chunk
    assert n_chunks * chunk == rows_per_w
    n_groups = n_chunks // nbuf
    assert n_groups * nbuf == n_chunks
    g = 128
    n_g = chunk // g

    mesh = plsc.VectorSubcoreMesh(core_axis_name="c", subcore_axis_name="s")

    @functools.partial(
        pl.kernel,
        mesh=mesh,
        out_type=jax.ShapeDtypeStruct((_TOTAL, _DIM), jnp.float32),
        scratch_types=[
            pltpu.VMEM((rows_per_w,), jnp.int32),
        ] + [pltpu.VMEM((chunk, _DIM), jnp.float32) for _ in range(nbuf)]
          + [pltpu.SemaphoreType.DMA for _ in range(2 * nbuf)],
        compiler_params=pltpu.CompilerParams(use_tc_tiling_on_sc=False),
    )
    def emb(idx_hbm, table_hbm, out_hbm, idx_all, *bufs_and_sems):
        rows = bufs_and_sems[:nbuf]
        sem_g = bufs_and_sems[nbuf:2 * nbuf]
        sem_s = bufs_and_sems[2 * nbuf:]
        wid = lax.axis_index("s") * nc + lax.axis_index("c")
        base = wid * rows_per_w
        pltpu.sync_copy(idx_hbm.at[pl.ds(base, rows_per_w)], idx_all)

        def group(i, carry):
            s = i * nbuf
            handles = []
            for b in range(nbuf):
                cstart = (s + b) * chunk

                @pl.when(i >= 1)
                def _():
                    pltpu.make_async_copy(
                        rows[b], out_hbm.at[pl.ds(base, chunk)], sem_s[b]
                    ).wait()

                hb = []
                for j in range(n_g):
                    hb.append(pltpu.async_copy(
                        table_hbm.at[idx_all.at[pl.ds(cstart + j * g, g)]],
                        rows[b].at[pl.ds(j * g, g)],
                        sem_g[b],
                    ))
                handles.append(hb)
            for b in range(nbuf):
                for c in handles[b]:
                    c.wait()
                pltpu.async_copy(
                    rows[b],
                    out_hbm.at[pl.ds(base + (s + b) * chunk, chunk)],
                    sem_s[b],
                )
            return carry

        lax.fori_loop(0, n_groups, group, 0)
        for b in range(nbuf):
            pltpu.make_async_copy(
                rows[b], out_hbm.at[pl.ds(base, chunk)], sem_s[b]
            ).wait()

    return emb


_emb = _build()


def kernel(input, table):
    idx_flat = input.reshape(_TOTAL)
    out = _emb(idx_flat, table)
    return out.reshape(_BATCH, _FIELDS, _DIM)
